# baseline (speedup 1.0000x reference)
import jax
import jax.numpy as jnp
from jax import lax
from jax.experimental import pallas as pl
from jax.experimental.pallas import tpu as pltpu
from jax.experimental.pallas import tpu_sc as plsc

_BATCH = 16384
_SEQ = 100
_OUT_DIM = 128
_ROWS = _SEQ * _BATCH
_NW = 32
_PER_W = _ROWS // _NW
_CHUNK = 512
_NCOPY = _PER_W // _CHUNK
_DEPTH = 4
_L = 16


def _sc_fill(emb_hbm, out_hbm, scal_v, buf_v, sems):
    c = lax.axis_index("c")
    s = lax.axis_index("s")
    wid = s * 2 + c

    pltpu.sync_copy(emb_hbm, scal_v)
    v = scal_v[...]

    def fill(r, carry):
        for k in range(_OUT_DIM // _L):
            buf_v[r, pl.ds(k * _L, _L)] = v
        return carry

    lax.fori_loop(0, _CHUNK, fill, 0)

    base = wid * _PER_W
    copies = []
    for i in range(_NCOPY):
        cp = pltpu.make_async_copy(
            buf_v,
            out_hbm.at[pl.ds(base + i * _CHUNK, _CHUNK)],
            sems.at[i % _DEPTH],
        )
        if i >= _DEPTH:
            copies[i - _DEPTH].wait()
        cp.start()
        copies.append(cp)
    for i in range(_NCOPY - _DEPTH, _NCOPY):
        copies[i].wait()


def kernel(inputs, emb_table):
    del inputs
    emb_flat = jnp.broadcast_to(emb_table.reshape((1,)), (_L,))
    out = pl.kernel(
        _sc_fill,
        out_type=jax.ShapeDtypeStruct((_ROWS, _OUT_DIM), jnp.float32),
        mesh=plsc.VectorSubcoreMesh(core_axis_name="c", subcore_axis_name="s"),
        scratch_types=[
            pltpu.VMEM((_L,), jnp.float32),
            pltpu.VMEM((_CHUNK, _OUT_DIM), jnp.float32),
            pltpu.SemaphoreType.DMA((_DEPTH,)),
        ],
    )(emb_flat)
    return jnp.transpose(out.reshape(_SEQ, _BATCH, _OUT_DIM), (1, 0, 2))


# speedup vs baseline: 1.1529x; 1.1529x over previous
import jax
import jax.numpy as jnp
from jax.experimental import pallas as pl
from jax.experimental.pallas import tpu as pltpu

_BATCH = 16384
_SEQ = 100
_OUT_DIM = 128
_BLOCK_S = 2


def _fill_block(emb_ref, out_ref):
    out_ref[...] = jnp.broadcast_to(emb_ref[0, 0], out_ref.shape)


def kernel(inputs, emb_table):
    del inputs
    out = pl.pallas_call(
        _fill_block,
        grid=(_SEQ // _BLOCK_S,),
        in_specs=[pl.BlockSpec((1, 1), lambda i: (0, 0))],
        out_specs=pl.BlockSpec((_BLOCK_S, _BATCH, _OUT_DIM), lambda i: (i, 0, 0)),
        out_shape=jax.ShapeDtypeStruct((_SEQ, _BATCH, _OUT_DIM), jnp.float32),
        compiler_params=pltpu.CompilerParams(
            dimension_semantics=("parallel",),
        ),
    )(emb_table)
    return jnp.transpose(out, (1, 0, 2))


# speedup vs baseline: 1.1604x; 1.0065x over previous
import jax
import jax.numpy as jnp
from jax.experimental import pallas as pl
from jax.experimental.pallas import tpu as pltpu

_BATCH = 16384
_SEQ = 100
_OUT_DIM = 128
_BLOCK_S = 1


def _fill_block(emb_ref, out_ref):
    out_ref[...] = jnp.broadcast_to(emb_ref[0, 0], out_ref.shape)


def kernel(inputs, emb_table):
    del inputs
    out = pl.pallas_call(
        _fill_block,
        grid=(_SEQ // _BLOCK_S,),
        in_specs=[pl.BlockSpec((1, 1), lambda i: (0, 0))],
        out_specs=pl.BlockSpec((_BLOCK_S, _BATCH, _OUT_DIM), lambda i: (i, 0, 0)),
        out_shape=jax.ShapeDtypeStruct((_SEQ, _BATCH, _OUT_DIM), jnp.float32),
        compiler_params=pltpu.CompilerParams(
            dimension_semantics=("parallel",),
        ),
    )(emb_table)
    return jnp.transpose(out, (1, 0, 2))
